# baseline (speedup 1.0000x reference)
import jax
import jax.numpy as jnp
from jax import lax
from jax.experimental import pallas as pl
from jax.experimental.pallas import tpu as pltpu
from jax.experimental.pallas import tpu_sc as plsc

NUM_POINTS = 4096
DIMS = 128
BATCH = 512

_NC = 2
_NS = 16
_NW = _NC * _NS
_BPW = BATCH // _NW
_L = 16


def _sc_gather_body(idx_hbm, graph_hbm, g_out,
                    idx_all, rows, gcols, sem_g, sem_g2):
    c = lax.axis_index("c")
    s = lax.axis_index("s")
    wid = s * _NC + c
    base = wid * _BPW
    half = _BPW // 2
    pltpu.sync_copy(idx_hbm, idx_all)
    cp_g0 = pltpu.async_copy(graph_hbm.at[idx_all.at[pl.ds(base, half)]],
                             rows.at[pl.ds(0, half)], sem_g)
    cp_g1 = pltpu.async_copy(
        graph_hbm.at[idx_all.at[pl.ds(base + half, half)]],
        rows.at[pl.ds(half, half)], sem_g2)

    def make_col_block(lo):
        def col_block(jb, carry):
            cols = idx_all[pl.ds(jb * _L, _L)]
            for li in range(lo, lo + half):
                rsel = jnp.full((_L,), li, jnp.int32)
                vals = plsc.load_gather(rows, [rsel, cols])
                gcols[li, pl.ds(jb * _L, _L)] = vals
            return carry
        return col_block

    cp_g0.wait()
    lax.fori_loop(0, BATCH // _L, make_col_block(0), 0)
    cp_g1.wait()
    lax.fori_loop(0, BATCH // _L, make_col_block(half), 0)

    pltpu.sync_copy(gcols, g_out.at[pl.ds(base, _BPW)])


_sc_gather = pl.kernel(
    _sc_gather_body,
    out_type=jax.ShapeDtypeStruct((BATCH, BATCH), jnp.float32),
    mesh=plsc.VectorSubcoreMesh(core_axis_name="c", subcore_axis_name="s"),
    compiler_params=pltpu.CompilerParams(needs_layout_passes=False),
    scratch_types=[
        pltpu.VMEM((BATCH,), jnp.int32),
        pltpu.VMEM((_BPW, NUM_POINTS), jnp.float32),
        pltpu.VMEM((_BPW, BATCH), jnp.float32),
        pltpu.SemaphoreType.DMA,
        pltpu.SemaphoreType.DMA,
    ],
)


def _tc_dist_body(idx_ref, emb_ref, d2_ref):
    iv = idx_ref[...]
    emb = emb_ref[...]
    oh = (lax.broadcasted_iota(jnp.int32, (BATCH, NUM_POINTS), 1)
          == iv[:, None]).astype(jnp.float32)
    x = lax.dot_general(oh, emb, (((1,), (0,)), ((), ())),
                        preferred_element_type=jnp.float32)
    xx = lax.dot_general(x, x, (((1,), (1,)), ((), ())),
                         preferred_element_type=jnp.float32)
    n2 = jnp.sum(x * x, axis=1)
    d2_ref[...] = n2[:, None] + n2[None, :] - 2.0 * xx + 1e-12


_tc_dist = pl.pallas_call(
    _tc_dist_body,
    out_shape=jax.ShapeDtypeStruct((BATCH, BATCH), jnp.float32),
)


def _tc_loss_body(d2_ref, g_ref, o_ref):
    d2 = d2_ref[...]
    g = g_ref[...]
    m = jnp.abs(d2 / (g * g) - 1.0)
    row = lax.broadcasted_iota(jnp.int32, (BATCH, BATCH), 0)
    col = lax.broadcasted_iota(jnp.int32, (BATCH, BATCH), 1)
    o_ref[0, 0] = jnp.sum(jnp.where(row < col, m, 0.0))


_tc_loss = pl.pallas_call(
    _tc_loss_body,
    out_shape=jax.ShapeDtypeStruct((1, 1), jnp.float32),
    out_specs=pl.BlockSpec(memory_space=pltpu.SMEM),
)


def kernel(input_index, embeds, graph):
    idx = input_index.astype(jnp.int32)
    gmat = _sc_gather(idx, graph)
    d2 = _tc_dist(idx, embeds)
    return _tc_loss(d2, gmat)[0, 0]


# speedup vs baseline: 1.0010x; 1.0010x over previous
import jax
import jax.numpy as jnp
from jax import lax
from jax.experimental import pallas as pl
from jax.experimental.pallas import tpu as pltpu
from jax.experimental.pallas import tpu_sc as plsc

NUM_POINTS = 4096
DIMS = 128
BATCH = 512

_NC = 2
_NS = 16
_NW = _NC * _NS
_BPW = BATCH // _NW
_L = 16


def _sc_gather_body(idx_hbm, graph_hbm, g_out,
                    idx_all, rows, gcols, sem_g, sem_g2):
    c = lax.axis_index("c")
    s = lax.axis_index("s")
    wid = s * _NC + c
    base = wid * _BPW
    half = _BPW // 2
    pltpu.sync_copy(idx_hbm, idx_all)
    cp_g0 = pltpu.async_copy(graph_hbm.at[idx_all.at[pl.ds(base, half)]],
                             rows.at[pl.ds(0, half)], sem_g)
    cp_g1 = pltpu.async_copy(
        graph_hbm.at[idx_all.at[pl.ds(base + half, half)]],
        rows.at[pl.ds(half, half)], sem_g2)

    def make_col_block(lo):
        def col_block(jb, carry):
            cols = idx_all[pl.ds(jb * _L, _L)]
            for li in range(lo, lo + half):
                rsel = jnp.full((_L,), li, jnp.int32)
                vals = plsc.load_gather(rows, [rsel, cols])
                gcols[li, pl.ds(jb * _L, _L)] = vals
            return carry
        return col_block

    cp_g0.wait()
    lax.fori_loop(0, BATCH // _L, make_col_block(0), 0)
    cp_g1.wait()
    lax.fori_loop(0, BATCH // _L, make_col_block(half), 0)

    pltpu.sync_copy(gcols, g_out.at[pl.ds(base, _BPW)])


_sc_gather = pl.kernel(
    _sc_gather_body,
    out_type=jax.ShapeDtypeStruct((BATCH, BATCH), jnp.float32),
    mesh=plsc.VectorSubcoreMesh(core_axis_name="c", subcore_axis_name="s"),
    compiler_params=pltpu.CompilerParams(needs_layout_passes=False,
                                         skip_device_barrier=True),
    scratch_types=[
        pltpu.VMEM((BATCH,), jnp.int32),
        pltpu.VMEM((_BPW, NUM_POINTS), jnp.float32),
        pltpu.VMEM((_BPW, BATCH), jnp.float32),
        pltpu.SemaphoreType.DMA,
        pltpu.SemaphoreType.DMA,
    ],
)


def _tc_dist_body(idx_ref, emb_ref, d2_ref):
    iv = idx_ref[...]
    emb = emb_ref[...]
    oh = (lax.broadcasted_iota(jnp.int32, (BATCH, NUM_POINTS), 1)
          == iv[:, None]).astype(jnp.float32)
    x = lax.dot_general(oh, emb, (((1,), (0,)), ((), ())),
                        preferred_element_type=jnp.float32)
    xx = lax.dot_general(x, x, (((1,), (1,)), ((), ())),
                         preferred_element_type=jnp.float32)
    n2 = jnp.sum(x * x, axis=1)
    d2_ref[...] = n2[:, None] + n2[None, :] - 2.0 * xx + 1e-12


_tc_dist = pl.pallas_call(
    _tc_dist_body,
    out_shape=jax.ShapeDtypeStruct((BATCH, BATCH), jnp.float32),
)


def _tc_loss_body(d2_ref, g_ref, o_ref):
    d2 = d2_ref[...]
    g = g_ref[...]
    m = jnp.abs(d2 / (g * g) - 1.0)
    row = lax.broadcasted_iota(jnp.int32, (BATCH, BATCH), 0)
    col = lax.broadcasted_iota(jnp.int32, (BATCH, BATCH), 1)
    o_ref[0, 0] = jnp.sum(jnp.where(row < col, m, 0.0))


_tc_loss = pl.pallas_call(
    _tc_loss_body,
    out_shape=jax.ShapeDtypeStruct((1, 1), jnp.float32),
    out_specs=pl.BlockSpec(memory_space=pltpu.SMEM),
)


def kernel(input_index, embeds, graph):
    idx = input_index.astype(jnp.int32)
    gmat = _sc_gather(idx, graph)
    d2 = _tc_dist(idx, embeds)
    return _tc_loss(d2, gmat)[0, 0]


# speedup vs baseline: 1.0137x; 1.0127x over previous
import jax
import jax.numpy as jnp
from jax import lax
from jax.experimental import pallas as pl
from jax.experimental.pallas import tpu as pltpu
from jax.experimental.pallas import tpu_sc as plsc

NUM_POINTS = 4096
DIMS = 128
BATCH = 512

_NC = 2
_NS = 16
_NW = _NC * _NS
_BPW = BATCH // _NW
_L = 16


def _sc_gather_body(idx_hbm, graph_hbm, g_out,
                    idx_all, rows, gcols, sem_g, sem_g2):
    c = lax.axis_index("c")
    s = lax.axis_index("s")
    wid = s * _NC + c
    base = wid * _BPW
    half = _BPW // 2
    pltpu.sync_copy(idx_hbm, idx_all)
    cp_g0 = pltpu.async_copy(graph_hbm.at[idx_all.at[pl.ds(base, half)]],
                             rows.at[pl.ds(0, half)], sem_g)
    cp_g1 = pltpu.async_copy(
        graph_hbm.at[idx_all.at[pl.ds(base + half, half)]],
        rows.at[pl.ds(half, half)], sem_g2)

    def make_col_block(lo):
        def col_block(jb, carry):
            cols = idx_all[pl.ds(jb * _L, _L)]
            for li in range(lo, lo + half):
                rsel = jnp.full((_L,), li, jnp.int32)
                vals = plsc.load_gather(rows, [rsel, cols])
                gcols[li, pl.ds(jb * _L, _L)] = vals
            return carry
        return col_block

    cp_g0.wait()
    lax.fori_loop(0, BATCH // _L, make_col_block(0), 0)
    cp_o0 = pltpu.async_copy(gcols.at[pl.ds(0, half)],
                             g_out.at[pl.ds(base, half)], sem_g)
    cp_g1.wait()
    lax.fori_loop(0, BATCH // _L, make_col_block(half), 0)
    cp_o1 = pltpu.async_copy(gcols.at[pl.ds(half, half)],
                             g_out.at[pl.ds(base + half, half)], sem_g2)
    cp_o0.wait()
    cp_o1.wait()


_sc_gather = pl.kernel(
    _sc_gather_body,
    out_type=jax.ShapeDtypeStruct((BATCH, BATCH), jnp.float32),
    mesh=plsc.VectorSubcoreMesh(core_axis_name="c", subcore_axis_name="s"),
    compiler_params=pltpu.CompilerParams(needs_layout_passes=False),
    scratch_types=[
        pltpu.VMEM((BATCH,), jnp.int32),
        pltpu.VMEM((_BPW, NUM_POINTS), jnp.float32),
        pltpu.VMEM((_BPW, BATCH), jnp.float32),
        pltpu.SemaphoreType.DMA,
        pltpu.SemaphoreType.DMA,
    ],
)


def _tc_dist_body(idx_ref, emb_ref, d2_ref):
    iv = idx_ref[...]
    emb = emb_ref[...]
    oh = (lax.broadcasted_iota(jnp.int32, (BATCH, NUM_POINTS), 1)
          == iv[:, None]).astype(jnp.float32)
    x = lax.dot_general(oh, emb, (((1,), (0,)), ((), ())),
                        preferred_element_type=jnp.float32)
    xx = lax.dot_general(x, x, (((1,), (1,)), ((), ())),
                         preferred_element_type=jnp.float32)
    n2 = jnp.sum(x * x, axis=1)
    d2 = n2[:, None] + n2[None, :] - 2.0 * xx + 1e-12
    d2_ref[...] = d2.astype(jnp.bfloat16)


_tc_dist = pl.pallas_call(
    _tc_dist_body,
    out_shape=jax.ShapeDtypeStruct((BATCH, BATCH), jnp.bfloat16),
)


def _tc_loss_body(d2_ref, g_ref, o_ref):
    d2 = d2_ref[...].astype(jnp.float32)
    g = g_ref[...]
    m = jnp.abs(d2 / (g * g) - 1.0)
    row = lax.broadcasted_iota(jnp.int32, (BATCH, BATCH), 0)
    col = lax.broadcasted_iota(jnp.int32, (BATCH, BATCH), 1)
    o_ref[0, 0] = jnp.sum(jnp.where(row < col, m, 0.0))


_tc_loss = pl.pallas_call(
    _tc_loss_body,
    out_shape=jax.ShapeDtypeStruct((1, 1), jnp.float32),
    out_specs=pl.BlockSpec(memory_space=pltpu.SMEM),
)


def kernel(input_index, embeds, graph):
    idx = input_index.astype(jnp.int32)
    gmat = _sc_gather(idx, graph)
    d2 = _tc_dist(idx, embeds)
    return _tc_loss(d2, gmat)[0, 0]


# speedup vs baseline: 1.0140x; 1.0003x over previous
import jax
import jax.numpy as jnp
from jax import lax
from jax.experimental import pallas as pl
from jax.experimental.pallas import tpu as pltpu
from jax.experimental.pallas import tpu_sc as plsc

NUM_POINTS = 4096
DIMS = 128
BATCH = 512

_NC = 2
_NS = 16
_NW = _NC * _NS
_BPW = BATCH // _NW
_L = 16


def _sc_gather_body(idx_hbm, graph_hbm, g_out,
                    idx_all, rows, gcols, *sems):
    c = lax.axis_index("c")
    s = lax.axis_index("s")
    wid = s * _NC + c
    base = wid * _BPW
    nchunk = 2
    rpc = _BPW // nchunk
    pltpu.sync_copy(idx_hbm, idx_all)
    cps = []
    for ch in range(nchunk):
        cps.append(pltpu.async_copy(
            graph_hbm.at[idx_all.at[pl.ds(base + ch * rpc, rpc)]],
            rows.at[pl.ds(ch * rpc, rpc)], sems[ch]))

    def make_col_block(lo):
        def col_block(jb, carry):
            cols = idx_all[pl.ds(jb * _L, _L)]
            for li in range(lo, lo + rpc):
                rsel = jnp.full((_L,), li, jnp.int32)
                vals = plsc.load_gather(rows, [rsel, cols])
                gcols[li, pl.ds(jb * _L, _L)] = vals
            return carry
        return col_block

    outs = []
    for ch in range(nchunk):
        cps[ch].wait()
        lax.fori_loop(0, BATCH // _L, make_col_block(ch * rpc), 0)
        outs.append(pltpu.async_copy(
            gcols.at[pl.ds(ch * rpc, rpc)],
            g_out.at[pl.ds(base + ch * rpc, rpc)], sems[ch]))
    for cp in outs:
        cp.wait()


_sc_gather = pl.kernel(
    _sc_gather_body,
    out_type=jax.ShapeDtypeStruct((BATCH, BATCH), jnp.float32),
    mesh=plsc.VectorSubcoreMesh(core_axis_name="c", subcore_axis_name="s"),
    compiler_params=pltpu.CompilerParams(needs_layout_passes=False),
    scratch_types=[
        pltpu.VMEM((BATCH,), jnp.int32),
        pltpu.VMEM((_BPW, NUM_POINTS), jnp.float32),
        pltpu.VMEM((_BPW, BATCH), jnp.float32),
        pltpu.SemaphoreType.DMA,
        pltpu.SemaphoreType.DMA,
        pltpu.SemaphoreType.DMA,
        pltpu.SemaphoreType.DMA,
    ],
)


def _tc_dist_body(idx_ref, emb_ref, d2_ref):
    iv = idx_ref[...]
    emb = emb_ref[...]
    oh = (lax.broadcasted_iota(jnp.int32, (BATCH, NUM_POINTS), 1)
          == iv[:, None]).astype(jnp.float32)
    x = lax.dot_general(oh, emb, (((1,), (0,)), ((), ())),
                        preferred_element_type=jnp.float32)
    xx = lax.dot_general(x, x, (((1,), (1,)), ((), ())),
                         preferred_element_type=jnp.float32)
    n2 = jnp.sum(x * x, axis=1)
    d2 = n2[:, None] + n2[None, :] - 2.0 * xx + 1e-12
    d2_ref[...] = d2.astype(jnp.bfloat16)


_tc_dist = pl.pallas_call(
    _tc_dist_body,
    out_shape=jax.ShapeDtypeStruct((BATCH, BATCH), jnp.bfloat16),
)


def _tc_loss_body(d2_ref, g_ref, o_ref):
    d2 = d2_ref[...].astype(jnp.float32)
    g = g_ref[...]
    m = jnp.abs(d2 / (g * g) - 1.0)
    row = lax.broadcasted_iota(jnp.int32, (BATCH, BATCH), 0)
    col = lax.broadcasted_iota(jnp.int32, (BATCH, BATCH), 1)
    o_ref[0, 0] = jnp.sum(jnp.where(row < col, m, 0.0))


_tc_loss = pl.pallas_call(
    _tc_loss_body,
    out_shape=jax.ShapeDtypeStruct((1, 1), jnp.float32),
    out_specs=pl.BlockSpec(memory_space=pltpu.SMEM),
)


def kernel(input_index, embeds, graph):
    idx = input_index.astype(jnp.int32)
    gmat = _sc_gather(idx, graph)
    d2 = _tc_dist(idx, embeds)
    return _tc_loss(d2, gmat)[0, 0]


# speedup vs baseline: 1.0159x; 1.0018x over previous
import jax
import jax.numpy as jnp
from jax import lax
from jax.experimental import pallas as pl
from jax.experimental.pallas import tpu as pltpu
from jax.experimental.pallas import tpu_sc as plsc

NUM_POINTS = 4096
DIMS = 128
BATCH = 512

_NC = 2
_NS = 16
_NW = _NC * _NS
_BPW = BATCH // _NW
_L = 16


def _sc_gather_body(idx_hbm, graph_hbm, g_out,
                    idx_all, rows, gcols, *sems):
    c = lax.axis_index("c")
    s = lax.axis_index("s")
    wid = s * _NC + c
    base = wid * _BPW
    nchunk = 2
    rpc = _BPW // nchunk
    pltpu.sync_copy(idx_hbm, idx_all)
    cps = []
    for ch in range(nchunk):
        cps.append(pltpu.async_copy(
            graph_hbm.at[idx_all.at[pl.ds(base + ch * rpc, rpc)]],
            rows.at[pl.ds(ch * rpc, rpc)], sems[ch]))

    def make_col_block(lo):
        def col_block(jb, carry):
            cols = idx_all[pl.ds(jb * _L, _L)]
            for li in range(lo, lo + rpc):
                rsel = jnp.full((_L,), li, jnp.int32)
                vals = plsc.load_gather(rows, [rsel, cols])
                gcols[li, pl.ds(jb * _L, _L)] = vals
            return carry
        return col_block

    outs = []
    for ch in range(nchunk):
        cps[ch].wait()
        lax.fori_loop(0, BATCH // _L, make_col_block(ch * rpc), 0)
        outs.append(pltpu.async_copy(
            gcols.at[pl.ds(ch * rpc, rpc)],
            g_out.at[pl.ds(base + ch * rpc, rpc)], sems[ch]))
    for cp in outs:
        cp.wait()


_sc_gather = pl.kernel(
    _sc_gather_body,
    out_type=jax.ShapeDtypeStruct((BATCH, BATCH), jnp.float32),
    mesh=plsc.VectorSubcoreMesh(core_axis_name="c", subcore_axis_name="s"),
    compiler_params=pltpu.CompilerParams(needs_layout_passes=False),
    scratch_types=[
        pltpu.VMEM((BATCH,), jnp.int32),
        pltpu.VMEM((_BPW, NUM_POINTS), jnp.float32),
        pltpu.VMEM((_BPW, BATCH), jnp.float32),
        pltpu.SemaphoreType.DMA,
        pltpu.SemaphoreType.DMA,
        pltpu.SemaphoreType.DMA,
        pltpu.SemaphoreType.DMA,
    ],
)


def _tc_dist_body(idx_ref, emb_ref, d2_ref):
    iv = idx_ref[...]
    emb = emb_ref[...]
    oh = (lax.broadcasted_iota(jnp.int32, (BATCH, NUM_POINTS), 1)
          == iv[:, None]).astype(jnp.float32)
    x = lax.dot_general(oh, emb, (((1,), (0,)), ((), ())),
                        preferred_element_type=jnp.float32)
    xx = lax.dot_general(x, x, (((1,), (1,)), ((), ())),
                         preferred_element_type=jnp.float32)
    n2 = jnp.sum(x * x, axis=1)
    d2 = n2[:, None] + n2[None, :] - 2.0 * xx + 1e-12
    d2_ref[...] = d2.astype(jnp.bfloat16)


_tc_dist = pl.pallas_call(
    _tc_dist_body,
    out_shape=jax.ShapeDtypeStruct((BATCH, BATCH), jnp.bfloat16),
)


def _tc_loss_body(d2_ref, g_ref, o_ref):
    d2 = d2_ref[...].astype(jnp.float32)
    g = g_ref[...]
    m = jnp.abs(d2 / (g * g) - 1.0)
    row = lax.broadcasted_iota(jnp.int32, (BATCH, BATCH), 0)
    col = lax.broadcasted_iota(jnp.int32, (BATCH, BATCH), 1)
    o_ref[0, 0] = jnp.sum(jnp.where(row < col, m, 0.0))


_tc_loss = pl.pallas_call(
    _tc_loss_body,
    out_shape=jax.ShapeDtypeStruct((1, 1), jnp.float32),
    out_specs=pl.BlockSpec(memory_space=pltpu.SMEM),
)


def kernel(input_index, embeds, graph):
    idx = input_index.astype(jnp.int32)
    d2 = _tc_dist(idx, embeds)
    gmat = _sc_gather(idx, graph)
    return _tc_loss(d2, gmat)[0, 0]
